# baseline (speedup 1.0000x reference)
import functools

import jax
import jax.numpy as jnp
from jax import lax
from jax.experimental import pallas as pl
from jax.experimental.pallas import tpu as pltpu
from jax.experimental.pallas import tpu_sc as plsc

B = 4096
DIM = 128
DESC_DIM = 768
NNB = 8
NW = 32
PB = B // NW
BLK = 128
NB = B // BLK

N_IDX = PB * (1 + NNB + NNB * NNB) + 16
N_IDXT = PB * (NNB + NNB * NNB)
N_REL = PB * (NNB + NNB * NNB) + 16


def _sc_body(ui_hbm, ii_hbm, adj_cat, ent, desc_tab,
             rel1_u, rel2_u, rel1_i, rel2_i,
             e0_u, e1_u, e2_u, e0_i, e1_i, e2_i, d_u, d_i,
             idxall, relbuf, abuf, ebuf, dbuf, sem, wsem):
    nc = 2
    wid = lax.axis_index("s") * nc + lax.axis_index("c")
    base = pl.multiple_of(wid * PB, PB)

    def extract_nbr(dst0):
        def ex(j, _):
            v = abuf[j, :]
            idxall[pl.ds(pl.multiple_of(dst0 + 8 * j, 8), 16)] = v
            return 0
        lax.fori_loop(0, PB, ex, 0)

    def extract_rel(dst0):
        def ex(q, _):
            j = PB - 1 - q
            v = abuf[j, :]
            relbuf[pl.ds(pl.multiple_of(dst0 + 8 * j, 8), 16)] = v
            return 0
        lax.fori_loop(0, PB, ex, 0)

    def do_side(base_hbm, rel1_o, rel2_o, e0_o, e1_o, e2_o, d_o):
        pltpu.sync_copy(base_hbm.at[pl.ds(base, PB)], idxall.at[pl.ds(0, PB)])
        pltpu.async_copy(adj_cat.at[idxall.at[pl.ds(0, PB)]], abuf, sem).wait()
        extract_nbr(PB)
        extract_rel(0)
        pltpu.sync_copy(
            relbuf.at[pl.ds(8, PB * NNB)],
            rel1_o.at[pl.ds(pl.multiple_of(wid * PB * NNB, PB), PB * NNB)])

        def hop2(c, _):
            off = pl.multiple_of(PB + PB * c, PB)
            pltpu.async_copy(adj_cat.at[idxall.at[pl.ds(off, PB)]], abuf,
                             sem).wait()
            extract_nbr(PB * (1 + NNB) + 1024 * c)
            extract_rel(1040)
            pltpu.sync_copy(
                relbuf.at[pl.ds(1048, PB * NNB)],
                rel2_o.at[pl.ds(pl.multiple_of(
                    wid * PB * NNB * NNB + PB * NNB * c, PB), PB * NNB)])
            return 0
        lax.fori_loop(0, NNB, hop2, 0)

        pltpu.async_copy(ent.at[idxall.at[pl.ds(0, PB)]], ebuf, sem).wait()
        pltpu.sync_copy(ebuf, e0_o.at[pl.ds(base, PB)])

        def g1(c, _):
            off = pl.multiple_of(PB + PB * c, PB)
            pltpu.async_copy(ent.at[idxall.at[pl.ds(off, PB)]], ebuf,
                             sem).wait()
            hs = [pltpu.async_copy(ebuf.at[pl.ds(NNB * t, NNB), :],
                                   e1_o.at[:, base + 16 * c + t, :], wsem)
                  for t in range(16)]
            for h in hs:
                h.wait()
            return 0
        lax.fori_loop(0, NNB, g1, 0)

        def g2(c, _):
            off = pl.multiple_of(PB * (1 + NNB) + PB * c, PB)
            pltpu.async_copy(ent.at[idxall.at[pl.ds(off, PB)]], ebuf,
                             sem).wait()
            h0 = pltpu.async_copy(ebuf.at[pl.ds(0, 64), :],
                                  e2_o.at[:, base + 2 * c, :], wsem)
            h1 = pltpu.async_copy(ebuf.at[pl.ds(64, 64), :],
                                  e2_o.at[:, base + 2 * c + 1, :], wsem)
            h0.wait()
            h1.wait()
            return 0
        lax.fori_loop(0, NNB * NNB, g2, 0)

        def gd(c, _):
            off = pl.multiple_of(16 * c, 16)
            pltpu.async_copy(desc_tab.at[idxall.at[pl.ds(off, 16)]], dbuf,
                             sem).wait()
            dst = pl.multiple_of(wid * PB + 16 * c, 16)
            pltpu.sync_copy(dbuf, d_o.at[pl.ds(dst, 16)])
            return 0
        lax.fori_loop(0, PB // 16, gd, 0)

    do_side(ui_hbm, rel1_u, rel2_u, e0_u, e1_u, e2_u, d_u)
    do_side(ii_hbm, rel1_i, rel2_i, e0_i, e1_i, e2_i, d_i)


def _sc_gather(user_index, item_index, adj_cat, ent, desc_tab):
    i32, f32 = jnp.int32, jnp.float32
    out_type = [
        jax.ShapeDtypeStruct((B * NNB,), i32),
        jax.ShapeDtypeStruct((B * NNB * NNB,), i32),
        jax.ShapeDtypeStruct((B * NNB,), i32),
        jax.ShapeDtypeStruct((B * NNB * NNB,), i32),
        jax.ShapeDtypeStruct((B, DIM), f32),
        jax.ShapeDtypeStruct((NNB, B, DIM), f32),
        jax.ShapeDtypeStruct((NNB * NNB, B, DIM), f32),
        jax.ShapeDtypeStruct((B, DIM), f32),
        jax.ShapeDtypeStruct((NNB, B, DIM), f32),
        jax.ShapeDtypeStruct((NNB * NNB, B, DIM), f32),
        jax.ShapeDtypeStruct((B, DESC_DIM), f32),
        jax.ShapeDtypeStruct((B, DESC_DIM), f32),
    ]
    scratch = [
        pltpu.VMEM((N_IDX,), i32),
        pltpu.VMEM((N_REL,), i32),
        pltpu.VMEM((PB, 2 * NNB), i32),
        pltpu.VMEM((PB, DIM), f32),
        pltpu.VMEM((16, DESC_DIM), f32),
        pltpu.SemaphoreType.DMA,
        pltpu.SemaphoreType.DMA,
    ]
    mesh = plsc.VectorSubcoreMesh(core_axis_name="c", subcore_axis_name="s")
    fn = pl.kernel(_sc_body, out_type=out_type, mesh=mesh,
                   scratch_types=scratch,
                   compiler_params=pltpu.CompilerParams(
                       use_tc_tiling_on_sc=False))
    return fn(user_index, item_index, adj_cat, ent, desc_tab)


def _tc_body(e0u, e1u, e2u, e0i, e1i, e2i, r1u, r2u, r1i, r2i, du, di,
             rel, wagg, bagg, wdr, bdr, w1, b1, w2, b2, w3, b3, wnm, bnm,
             og, od):
    wt = wagg[0:DIM, :]
    wb = wagg[DIM:2 * DIM, :]
    ba = bagg[...]
    rel_m = rel[...]

    def gather_scores(s_rel, idx):
        out = jnp.zeros(idx.shape, jnp.float32)
        for k in range(32):
            out = out + jnp.where(idx == k, s_rel[:, k:k + 1], 0.0)
        return out

    def softmax_rows(s):
        m = jnp.max(s, axis=-1, keepdims=True)
        e = jnp.exp(s - m)
        return e / jnp.sum(e, axis=-1, keepdims=True)

    def aggregate(side, e0, e1_ref, e2_ref, r1, r2):
        s_rel = lax.dot_general(side, rel_m, (((1,), (1,)), ((), ())),
                                preferred_element_type=jnp.float32)
        s2 = gather_scores(s_rel, r2)
        e1s = [e1_ref[i, :, :] for i in range(NNB)]
        aggs = []
        for i in range(NNB):
            w2g = softmax_rows(s2[:, NNB * i:NNB * (i + 1)])
            a = w2g[:, 0:1] * e2_ref[NNB * i, :, :]
            for n in range(1, NNB):
                a = a + w2g[:, n:n + 1] * e2_ref[NNB * i + n, :, :]
            aggs.append(a)
        x1 = jnp.concatenate(e1s, axis=0)
        a1 = jnp.concatenate(aggs, axis=0)
        h1 = jax.nn.sigmoid(
            jnp.dot(x1, wt, preferred_element_type=jnp.float32)
            + jnp.dot(a1, wb, preferred_element_type=jnp.float32) + ba)
        w1a = softmax_rows(gather_scores(s_rel, r1))
        agg1 = w1a[:, 0:1] * e1s[0]
        aggf = w1a[:, 0:1] * h1[0:BLK, :]
        for i in range(1, NNB):
            agg1 = agg1 + w1a[:, i:i + 1] * e1s[i]
            aggf = aggf + w1a[:, i:i + 1] * h1[BLK * i:BLK * (i + 1), :]
        h0 = jax.nn.sigmoid(
            jnp.dot(e0, wt, preferred_element_type=jnp.float32)
            + jnp.dot(agg1, wb, preferred_element_type=jnp.float32) + ba)
        return jnp.tanh(
            jnp.dot(h0, wt, preferred_element_type=jnp.float32)
            + jnp.dot(aggf, wb, preferred_element_type=jnp.float32) + ba)

    side_u = e0u[...]
    item_graph = aggregate(side_u, e0i[...], e1i, e2i, r1i[...], r2i[...])
    user_graph = aggregate(item_graph, side_u, e1u, e2u, r1u[...], r2u[...])
    og[...] = jax.nn.sigmoid(jnp.sum(user_graph * item_graph, axis=1))

    ud = jax.nn.relu(jnp.dot(du[...], wdr[...],
                             preferred_element_type=jnp.float32) + bdr[...])
    idd = jax.nn.relu(jnp.dot(di[...], wdr[...],
                              preferred_element_type=jnp.float32) + bdr[...])
    nl = jax.nn.relu(
        jnp.dot(ud, w1[0:DIM, :], preferred_element_type=jnp.float32)
        + jnp.dot(idd, w1[DIM:2 * DIM, :], preferred_element_type=jnp.float32)
        + b1[...])
    nl = jax.nn.relu(jnp.dot(nl, w2[...],
                             preferred_element_type=jnp.float32) + b2[...])
    nl = jax.nn.relu(jnp.dot(nl, w3[...],
                             preferred_element_type=jnp.float32) + b3[...])
    lmul = ud * idd
    sd = (jnp.sum(lmul * wnm[:, 0:DIM], axis=1)
          + jnp.sum(nl * wnm[:, DIM:DIM + DIM // 2], axis=1) + bnm[0, 0])
    od[...] = jax.nn.sigmoid(sd)


def _tc_compute(e0u, e1u, e2u, e0i, e1i, e2i, r1u, r2u, r1i, r2i, du, di,
                rel, wagg, bagg, wdr, bdr, w1, b1, w2, b2, w3, b3, wnm, bnm):
    f32 = jnp.float32

    def blk(shape, imap):
        return pl.BlockSpec(shape, imap)

    row = lambda i: (i, 0)
    whole = lambda i: (0, 0)
    nmaj = lambda i: (0, i, 0)
    in_specs = [
        blk((BLK, DIM), row), blk((NNB, BLK, DIM), nmaj),
        blk((NNB * NNB, BLK, DIM), nmaj),
        blk((BLK, DIM), row), blk((NNB, BLK, DIM), nmaj),
        blk((NNB * NNB, BLK, DIM), nmaj),
        blk((BLK, NNB), row), blk((BLK, NNB * NNB), row),
        blk((BLK, NNB), row), blk((BLK, NNB * NNB), row),
        blk((BLK, DESC_DIM), row), blk((BLK, DESC_DIM), row),
        blk((32, DIM), whole), blk((2 * DIM, DIM), whole),
        blk((1, DIM), whole), blk((DESC_DIM, DIM), whole),
        blk((1, DIM), whole), blk((2 * DIM, 2 * DIM), whole),
        blk((1, 2 * DIM), whole), blk((2 * DIM, DIM), whole),
        blk((1, DIM), whole), blk((DIM, DIM // 2), whole),
        blk((1, DIM // 2), whole), blk((1, DIM + DIM // 2), whole),
        blk((1, 1), whole),
    ]
    out_specs = [pl.BlockSpec((BLK,), lambda i: (i,)),
                 pl.BlockSpec((BLK,), lambda i: (i,))]
    out_shape = [jax.ShapeDtypeStruct((B,), f32),
                 jax.ShapeDtypeStruct((B,), f32)]
    return pl.pallas_call(
        _tc_body, grid=(NB,), in_specs=in_specs, out_specs=out_specs,
        out_shape=out_shape,
    )(e0u, e1u, e2u, e0i, e1i, e2i, r1u, r2u, r1i, r2i, du, di,
      rel, wagg, bagg, wdr, bdr, w1, b1, w2, b2, w3, b3, wnm, bnm)


def kernel(user_index, item_index, adj_ent, adj_rel, ent, rel, desc_tab,
           W_agg, b_agg, W_dr, b_dr, W1, b1, W2, b2, W3, b3, W_nm, b_nm):
    adj_cat = jnp.concatenate([adj_ent, adj_rel], axis=1)
    (rel1_u, rel2_u, rel1_i, rel2_i,
     e0u, e1u, e2u, e0i, e1i, e2i, du, di) = _sc_gather(
        user_index, item_index, adj_cat, ent, desc_tab)
    r1u = rel1_u.reshape(B, NNB)
    r2u = rel2_u.reshape(B, NNB * NNB)
    r1i = rel1_i.reshape(B, NNB)
    r2i = rel2_i.reshape(B, NNB * NNB)
    og, od = _tc_compute(
        e0u, e1u, e2u, e0i, e1i, e2i, r1u, r2u, r1i, r2i, du, di,
        rel, W_agg, b_agg.reshape(1, DIM), W_dr, b_dr.reshape(1, DIM),
        W1, b1.reshape(1, 2 * DIM), W2, b2.reshape(1, DIM),
        W3, b3.reshape(1, DIM // 2), W_nm.reshape(1, DIM + DIM // 2),
        b_nm.reshape(1, 1))
    return og, od


# speedup vs baseline: 1.2454x; 1.2454x over previous
import functools

import jax
import jax.numpy as jnp
from jax import lax
from jax.experimental import pallas as pl
from jax.experimental.pallas import tpu as pltpu
from jax.experimental.pallas import tpu_sc as plsc

B = 4096
DIM = 128
DESC_DIM = 768
NNB = 8
NW = 32
BLK = 128
NCH = 2


def _make_sc_gather(bc):
    pb = bc // NW

    def sc_body(ui_hbm, ii_hbm, adj_cat, ent, desc_tab,
                rel1_u, rel2_u, rel1_i, rel2_i,
                e0_u, e1_u, e2_u, e0_i, e1_i, e2_i, d_u, d_i,
                idxall, relbuf, abuf, ebuf, dbuf, sem, wsem):
        nc = 2
        wid = lax.axis_index("s") * nc + lax.axis_index("c")
        base = pl.multiple_of(wid * pb, pb)

        def extract(dst_ref, dst0, col):
            def ex(j, _):
                v = abuf[j, col:col + 16]
                dst_ref[pl.ds(pl.multiple_of(dst0 + 8 * j, 8), 16)] = v
                return 0
            lax.fori_loop(0, pb, ex, 0)

        def do_side(base_hbm, rel1_o, rel2_o, e0_o, e1_o, e2_o, d_o):
            pltpu.sync_copy(base_hbm.at[pl.ds(base, pb)],
                            idxall.at[pl.ds(0, pb)])
            pltpu.async_copy(adj_cat.at[idxall.at[pl.ds(0, pb)]], abuf,
                             sem).wait()
            extract(idxall, pb, 0)
            extract(relbuf, 0, 8)

            def hop2(c, _):
                off = pl.multiple_of(pb + pb * c, pb)
                pltpu.async_copy(adj_cat.at[idxall.at[pl.ds(off, pb)]],
                                 abuf, sem).wait()
                extract(idxall, pb * (1 + NNB) + pb * NNB * c, 0)
                extract(relbuf, pb * NNB + pb * NNB * c, 8)
                return 0
            lax.fori_loop(0, NNB, hop2, 0)
            pltpu.sync_copy(
                relbuf.at[pl.ds(0, pb * NNB)],
                rel1_o.at[pl.ds(pl.multiple_of(wid * pb * NNB, pb),
                                pb * NNB)])
            pltpu.sync_copy(
                relbuf.at[pl.ds(pb * NNB, pb * NNB * NNB)],
                rel2_o.at[pl.ds(pl.multiple_of(wid * pb * NNB * NNB, pb),
                                pb * NNB * NNB)])

            pltpu.async_copy(ent.at[idxall.at[pl.ds(0, pb)]],
                             ebuf.at[pl.ds(0, pb), :], sem).wait()
            pltpu.sync_copy(ebuf.at[pl.ds(0, pb), :],
                            e0_o.at[pl.ds(base, pb)])

            def g1(c, _):
                off = pl.multiple_of(pb + 128 * c, 8)
                pltpu.async_copy(ent.at[idxall.at[pl.ds(off, 128)]], ebuf,
                                 sem).wait()
                hs = [pltpu.async_copy(ebuf.at[pl.ds(NNB * t, NNB), :],
                                       e1_o.at[:, base + 16 * c + t, :],
                                       wsem)
                      for t in range(16)]
                for h in hs:
                    h.wait()
                return 0
            lax.fori_loop(0, pb // 16, g1, 0)

            def g2(c, _):
                off = pl.multiple_of(pb * (1 + NNB) + 128 * c, 8)
                pltpu.async_copy(ent.at[idxall.at[pl.ds(off, 128)]], ebuf,
                                 sem).wait()
                h0 = pltpu.async_copy(ebuf.at[pl.ds(0, 64), :],
                                      e2_o.at[:, base + 2 * c, :], wsem)
                h1 = pltpu.async_copy(ebuf.at[pl.ds(64, 64), :],
                                      e2_o.at[:, base + 2 * c + 1, :], wsem)
                h0.wait()
                h1.wait()
                return 0
            lax.fori_loop(0, pb // 2, g2, 0)

            def gd(c, _):
                off = pl.multiple_of(16 * c, 16)
                pltpu.async_copy(desc_tab.at[idxall.at[pl.ds(off, 16)]],
                                 dbuf, sem).wait()
                dst = pl.multiple_of(wid * pb + 16 * c, 16)
                pltpu.sync_copy(dbuf, d_o.at[pl.ds(dst, 16)])
                return 0
            lax.fori_loop(0, pb // 16, gd, 0)

        do_side(ui_hbm, rel1_u, rel2_u, e0_u, e1_u, e2_u, d_u)
        do_side(ii_hbm, rel1_i, rel2_i, e0_i, e1_i, e2_i, d_i)

    i32, f32 = jnp.int32, jnp.float32
    out_type = [
        jax.ShapeDtypeStruct((bc * NNB,), i32),
        jax.ShapeDtypeStruct((bc * NNB * NNB,), i32),
        jax.ShapeDtypeStruct((bc * NNB,), i32),
        jax.ShapeDtypeStruct((bc * NNB * NNB,), i32),
        jax.ShapeDtypeStruct((bc, DIM), f32),
        jax.ShapeDtypeStruct((NNB, bc, DIM), f32),
        jax.ShapeDtypeStruct((NNB * NNB, bc, DIM), f32),
        jax.ShapeDtypeStruct((bc, DIM), f32),
        jax.ShapeDtypeStruct((NNB, bc, DIM), f32),
        jax.ShapeDtypeStruct((NNB * NNB, bc, DIM), f32),
        jax.ShapeDtypeStruct((bc, DESC_DIM), f32),
        jax.ShapeDtypeStruct((bc, DESC_DIM), f32),
    ]
    scratch = [
        pltpu.VMEM((pb * (1 + NNB + NNB * NNB) + 16,), i32),
        pltpu.VMEM((pb * (NNB + NNB * NNB) + 16,), i32),
        pltpu.VMEM((pb, 128), i32),
        pltpu.VMEM((128, DIM), f32),
        pltpu.VMEM((16, DESC_DIM), f32),
        pltpu.SemaphoreType.DMA,
        pltpu.SemaphoreType.DMA,
    ]
    mesh = plsc.VectorSubcoreMesh(core_axis_name="c", subcore_axis_name="s")
    return pl.kernel(sc_body, out_type=out_type, mesh=mesh,
                     scratch_types=scratch)


def _tc_body(e0u, e1u, e2u, e0i, e1i, e2i, r1u, r2u, r1i, r2i, du, di,
             rel, wagg, bagg, wdr, bdr, w1, b1, w2, b2, w3, b3, wnm, bnm,
             og, od):
    wt = wagg[0:DIM, :]
    wb = wagg[DIM:2 * DIM, :]
    ba = bagg[...]
    rel_m = rel[...]

    def gather_scores(s_rel, idx):
        out = jnp.zeros(idx.shape, jnp.float32)
        for k in range(32):
            out = out + jnp.where(idx == k, s_rel[:, k:k + 1], 0.0)
        return out

    def softmax_rows(s):
        m = jnp.max(s, axis=-1, keepdims=True)
        e = jnp.exp(s - m)
        return e / jnp.sum(e, axis=-1, keepdims=True)

    def aggregate(side, e0, e1_ref, e2_ref, r1, r2):
        s_rel = lax.dot_general(side, rel_m, (((1,), (1,)), ((), ())),
                                preferred_element_type=jnp.float32)
        s2 = gather_scores(s_rel, r2)
        e1s = [e1_ref[i, :, :] for i in range(NNB)]
        aggs = []
        for i in range(NNB):
            w2g = softmax_rows(s2[:, NNB * i:NNB * (i + 1)])
            a = w2g[:, 0:1] * e2_ref[NNB * i, :, :]
            for n in range(1, NNB):
                a = a + w2g[:, n:n + 1] * e2_ref[NNB * i + n, :, :]
            aggs.append(a)
        x1 = jnp.concatenate(e1s, axis=0)
        a1 = jnp.concatenate(aggs, axis=0)
        h1 = jax.nn.sigmoid(
            jnp.dot(x1, wt, preferred_element_type=jnp.float32)
            + jnp.dot(a1, wb, preferred_element_type=jnp.float32) + ba)
        w1a = softmax_rows(gather_scores(s_rel, r1))
        agg1 = w1a[:, 0:1] * e1s[0]
        aggf = w1a[:, 0:1] * h1[0:BLK, :]
        for i in range(1, NNB):
            agg1 = agg1 + w1a[:, i:i + 1] * e1s[i]
            aggf = aggf + w1a[:, i:i + 1] * h1[BLK * i:BLK * (i + 1), :]
        h0 = jax.nn.sigmoid(
            jnp.dot(e0, wt, preferred_element_type=jnp.float32)
            + jnp.dot(agg1, wb, preferred_element_type=jnp.float32) + ba)
        return jnp.tanh(
            jnp.dot(h0, wt, preferred_element_type=jnp.float32)
            + jnp.dot(aggf, wb, preferred_element_type=jnp.float32) + ba)

    side_u = e0u[...]
    item_graph = aggregate(side_u, e0i[...], e1i, e2i, r1i[...], r2i[...])
    user_graph = aggregate(item_graph, side_u, e1u, e2u, r1u[...], r2u[...])
    og[...] = jax.nn.sigmoid(jnp.sum(user_graph * item_graph, axis=1))

    ud = jax.nn.relu(jnp.dot(du[...], wdr[...],
                             preferred_element_type=jnp.float32) + bdr[...])
    idd = jax.nn.relu(jnp.dot(di[...], wdr[...],
                              preferred_element_type=jnp.float32) + bdr[...])
    nl = jax.nn.relu(
        jnp.dot(ud, w1[0:DIM, :], preferred_element_type=jnp.float32)
        + jnp.dot(idd, w1[DIM:2 * DIM, :], preferred_element_type=jnp.float32)
        + b1[...])
    nl = jax.nn.relu(jnp.dot(nl, w2[...],
                             preferred_element_type=jnp.float32) + b2[...])
    nl = jax.nn.relu(jnp.dot(nl, w3[...],
                             preferred_element_type=jnp.float32) + b3[...])
    lmul = ud * idd
    sd = (jnp.sum(lmul * wnm[:, 0:DIM], axis=1)
          + jnp.sum(nl * wnm[:, DIM:DIM + DIM // 2], axis=1) + bnm[0, 0])
    od[...] = jax.nn.sigmoid(sd)


def _tc_compute(bc, e0u, e1u, e2u, e0i, e1i, e2i, r1u, r2u, r1i, r2i,
                du, di, rel, wagg, bagg, wdr, bdr, w1, b1, w2, b2, w3, b3,
                wnm, bnm):
    f32 = jnp.float32
    nb = bc // BLK

    def blk(shape, imap):
        return pl.BlockSpec(shape, imap)

    row = lambda i: (i, 0)
    whole = lambda i: (0, 0)
    nmaj = lambda i: (0, i, 0)
    in_specs = [
        blk((BLK, DIM), row), blk((NNB, BLK, DIM), nmaj),
        blk((NNB * NNB, BLK, DIM), nmaj),
        blk((BLK, DIM), row), blk((NNB, BLK, DIM), nmaj),
        blk((NNB * NNB, BLK, DIM), nmaj),
        blk((BLK, NNB), row), blk((BLK, NNB * NNB), row),
        blk((BLK, NNB), row), blk((BLK, NNB * NNB), row),
        blk((BLK, DESC_DIM), row), blk((BLK, DESC_DIM), row),
        blk((32, DIM), whole), blk((2 * DIM, DIM), whole),
        blk((1, DIM), whole), blk((DESC_DIM, DIM), whole),
        blk((1, DIM), whole), blk((2 * DIM, 2 * DIM), whole),
        blk((1, 2 * DIM), whole), blk((2 * DIM, DIM), whole),
        blk((1, DIM), whole), blk((DIM, DIM // 2), whole),
        blk((1, DIM // 2), whole), blk((1, DIM + DIM // 2), whole),
        blk((1, 1), whole),
    ]
    out_specs = [pl.BlockSpec((BLK,), lambda i: (i,)),
                 pl.BlockSpec((BLK,), lambda i: (i,))]
    out_shape = [jax.ShapeDtypeStruct((bc,), f32),
                 jax.ShapeDtypeStruct((bc,), f32)]
    return pl.pallas_call(
        _tc_body, grid=(nb,), in_specs=in_specs, out_specs=out_specs,
        out_shape=out_shape,
    )(e0u, e1u, e2u, e0i, e1i, e2i, r1u, r2u, r1i, r2i, du, di,
      rel, wagg, bagg, wdr, bdr, w1, b1, w2, b2, w3, b3, wnm, bnm)


def kernel(user_index, item_index, adj_ent, adj_rel, ent, rel, desc_tab,
           W_agg, b_agg, W_dr, b_dr, W1, b1, W2, b2, W3, b3, W_nm, b_nm):
    num_ent = adj_ent.shape[0]
    adj_cat = jnp.concatenate(
        [adj_ent, adj_rel,
         jnp.zeros((num_ent, 128 - 2 * NNB), jnp.int32)], axis=1)
    bc = B // NCH
    sc_fn = _make_sc_gather(bc)
    ogs, ods = [], []
    for k in range(NCH):
        ui = lax.dynamic_slice_in_dim(user_index, k * bc, bc)
        ii = lax.dynamic_slice_in_dim(item_index, k * bc, bc)
        (rel1_u, rel2_u, rel1_i, rel2_i,
         e0u, e1u, e2u, e0i, e1i, e2i, du, di) = sc_fn(
            ui, ii, adj_cat, ent, desc_tab)
        og, od = _tc_compute(
            bc, e0u, e1u, e2u, e0i, e1i, e2i,
            rel1_u.reshape(bc, NNB), rel2_u.reshape(bc, NNB * NNB),
            rel1_i.reshape(bc, NNB), rel2_i.reshape(bc, NNB * NNB),
            du, di, rel, W_agg, b_agg.reshape(1, DIM), W_dr,
            b_dr.reshape(1, DIM), W1, b1.reshape(1, 2 * DIM), W2,
            b2.reshape(1, DIM), W3, b3.reshape(1, DIM // 2),
            W_nm.reshape(1, DIM + DIM // 2), b_nm.reshape(1, 1))
        ogs.append(og)
        ods.append(od)
    return jnp.concatenate(ogs), jnp.concatenate(ods)


# speedup vs baseline: 1.3140x; 1.0550x over previous
import functools

import jax
import jax.numpy as jnp
from jax import lax
from jax.experimental import pallas as pl
from jax.experimental.pallas import tpu as pltpu
from jax.experimental.pallas import tpu_sc as plsc

B = 4096
DIM = 128
DESC_DIM = 768
NNB = 8
NW = 32
BLK = 128
NCH = 1


def _make_sc_gather(bc):
    pb = bc // NW

    def sc_body(ui_hbm, ii_hbm, adj_cat, ent, desc_tab,
                rel1_u, rel2_u, rel1_i, rel2_i,
                e0_u, e1_u, e2_u, e0_i, e1_i, e2_i, d_u, d_i,
                idxall, relbuf, abuf, ebuf, ebuf2, dbuf, sem, wsem):
        nc = 2
        wid = lax.axis_index("s") * nc + lax.axis_index("c")
        base = pl.multiple_of(wid * pb, pb)

        def extract(dst_ref, dst0, col):
            def ex(j, _):
                v = abuf[j, col:col + 16]
                dst_ref[pl.ds(pl.multiple_of(dst0 + 8 * j, 8), 16)] = v
                return 0
            lax.fori_loop(0, pb, ex, 0)

        def do_side(base_hbm, rel1_o, rel2_o, e0_o, e1_o, e2_o, d_o):
            pltpu.sync_copy(base_hbm.at[pl.ds(base, pb)],
                            idxall.at[pl.ds(0, pb)])
            pltpu.async_copy(adj_cat.at[idxall.at[pl.ds(0, pb)]], abuf,
                             sem).wait()
            extract(idxall, pb, 0)
            extract(relbuf, 0, 8)

            def hop2(c, _):
                off = pl.multiple_of(pb + pb * c, pb)
                pltpu.async_copy(adj_cat.at[idxall.at[pl.ds(off, pb)]],
                                 abuf, sem).wait()
                extract(idxall, pb * (1 + NNB) + pb * NNB * c, 0)
                extract(relbuf, pb * NNB + pb * NNB * c, 8)
                return 0
            lax.fori_loop(0, NNB, hop2, 0)
            pltpu.sync_copy(
                relbuf.at[pl.ds(0, pb * NNB)],
                rel1_o.at[pl.ds(pl.multiple_of(wid * pb * NNB, pb),
                                pb * NNB)])
            pltpu.sync_copy(
                relbuf.at[pl.ds(pb * NNB, pb * NNB * NNB)],
                rel2_o.at[pl.ds(pl.multiple_of(wid * pb * NNB * NNB, pb),
                                pb * NNB * NNB)])

            pltpu.async_copy(ent.at[idxall.at[pl.ds(0, pb)]],
                             ebuf.at[pl.ds(0, pb), :], sem).wait()
            pltpu.sync_copy(ebuf.at[pl.ds(0, pb), :],
                            e0_o.at[pl.ds(base, pb)])

            def drain2():
                pltpu.make_async_copy(ent.at[pl.ds(0, 128)], ebuf,
                                      wsem).wait()
                pltpu.make_async_copy(ent.at[pl.ds(0, 128)], ebuf2,
                                      wsem).wait()

            def g1(c2, _):
                @pl.when(c2 > 0)
                def _():
                    drain2()
                for k, buf in ((0, ebuf), (1, ebuf2)):
                    c = 2 * c2 + k
                    off = pl.multiple_of(pb + 128 * c, 8)
                    pltpu.async_copy(ent.at[idxall.at[pl.ds(off, 128)]],
                                     buf, sem).wait()
                    for t in range(16):
                        pltpu.async_copy(buf.at[pl.ds(NNB * t, NNB), :],
                                         e1_o.at[:, base + 16 * c + t, :],
                                         wsem)
                return 0
            lax.fori_loop(0, pb // 32, g1, 0)
            drain2()

            def g2(c2, _):
                @pl.when(c2 > 0)
                def _():
                    drain2()
                for k, buf in ((0, ebuf), (1, ebuf2)):
                    c = 2 * c2 + k
                    off = pl.multiple_of(pb * (1 + NNB) + 128 * c, 8)
                    pltpu.async_copy(ent.at[idxall.at[pl.ds(off, 128)]],
                                     buf, sem).wait()
                    pltpu.async_copy(buf.at[pl.ds(0, 64), :],
                                     e2_o.at[:, base + 2 * c, :], wsem)
                    pltpu.async_copy(buf.at[pl.ds(64, 64), :],
                                     e2_o.at[:, base + 2 * c + 1, :], wsem)
                return 0
            lax.fori_loop(0, pb // 4, g2, 0)
            drain2()

            def gd(c, _):
                off = pl.multiple_of(16 * c, 16)
                pltpu.async_copy(desc_tab.at[idxall.at[pl.ds(off, 16)]],
                                 dbuf, sem).wait()
                dst = pl.multiple_of(wid * pb + 16 * c, 16)
                pltpu.sync_copy(dbuf, d_o.at[pl.ds(dst, 16)])
                return 0
            lax.fori_loop(0, pb // 16, gd, 0)

        do_side(ui_hbm, rel1_u, rel2_u, e0_u, e1_u, e2_u, d_u)
        do_side(ii_hbm, rel1_i, rel2_i, e0_i, e1_i, e2_i, d_i)

    i32, f32, bf16 = jnp.int32, jnp.float32, jnp.bfloat16
    out_type = [
        jax.ShapeDtypeStruct((bc * NNB,), i32),
        jax.ShapeDtypeStruct((bc * NNB * NNB,), i32),
        jax.ShapeDtypeStruct((bc * NNB,), i32),
        jax.ShapeDtypeStruct((bc * NNB * NNB,), i32),
        jax.ShapeDtypeStruct((bc, DIM), f32),
        jax.ShapeDtypeStruct((NNB, bc, DIM), f32),
        jax.ShapeDtypeStruct((NNB * NNB, bc, DIM), f32),
        jax.ShapeDtypeStruct((bc, DIM), f32),
        jax.ShapeDtypeStruct((NNB, bc, DIM), f32),
        jax.ShapeDtypeStruct((NNB * NNB, bc, DIM), f32),
        jax.ShapeDtypeStruct((bc, DESC_DIM), f32),
        jax.ShapeDtypeStruct((bc, DESC_DIM), f32),
    ]
    scratch = [
        pltpu.VMEM((pb * (1 + NNB + NNB * NNB) + 16,), i32),
        pltpu.VMEM((pb * (NNB + NNB * NNB) + 16,), i32),
        pltpu.VMEM((pb, 128), i32),
        pltpu.VMEM((128, DIM), f32),
        pltpu.VMEM((128, DIM), f32),
        pltpu.VMEM((16, DESC_DIM), f32),
        pltpu.SemaphoreType.DMA,
        pltpu.SemaphoreType.DMA,
    ]
    mesh = plsc.VectorSubcoreMesh(core_axis_name="c", subcore_axis_name="s")
    return pl.kernel(sc_body, out_type=out_type, mesh=mesh,
                     scratch_types=scratch)


def _tc_body(e0u, e1u, e2u, e0i, e1i, e2i, r1u, r2u, r1i, r2i, du, di,
             rel, wagg, bagg, wdr, bdr, w1, b1, w2, b2, w3, b3, wnm, bnm,
             og, od):
    wt = wagg[0:DIM, :]
    wb = wagg[DIM:2 * DIM, :]
    ba = bagg[...]
    rel_m = rel[...]

    def gather_scores(s_rel, idx):
        out = jnp.zeros(idx.shape, jnp.float32)
        for k in range(32):
            out = out + jnp.where(idx == k, s_rel[:, k:k + 1], 0.0)
        return out

    def softmax_rows(s):
        m = jnp.max(s, axis=-1, keepdims=True)
        e = jnp.exp(s - m)
        return e / jnp.sum(e, axis=-1, keepdims=True)

    def aggregate(side, e0, e1_ref, e2_ref, r1, r2):
        s_rel = lax.dot_general(side, rel_m, (((1,), (1,)), ((), ())),
                                preferred_element_type=jnp.float32)
        s2 = gather_scores(s_rel, r2)
        e1s = [e1_ref[i, :, :].astype(jnp.float32) for i in range(NNB)]
        aggs = []
        for i in range(NNB):
            w2g = softmax_rows(s2[:, NNB * i:NNB * (i + 1)])
            a = w2g[:, 0:1] * e2_ref[NNB * i, :, :].astype(jnp.float32)
            for n in range(1, NNB):
                a = a + w2g[:, n:n + 1] * e2_ref[NNB * i + n, :, :].astype(jnp.float32)
            aggs.append(a)
        x1 = jnp.concatenate(e1s, axis=0)
        a1 = jnp.concatenate(aggs, axis=0)
        h1 = jax.nn.sigmoid(
            jnp.dot(x1, wt, preferred_element_type=jnp.float32)
            + jnp.dot(a1, wb, preferred_element_type=jnp.float32) + ba)
        w1a = softmax_rows(gather_scores(s_rel, r1))
        agg1 = w1a[:, 0:1] * e1s[0]
        aggf = w1a[:, 0:1] * h1[0:BLK, :]
        for i in range(1, NNB):
            agg1 = agg1 + w1a[:, i:i + 1] * e1s[i]
            aggf = aggf + w1a[:, i:i + 1] * h1[BLK * i:BLK * (i + 1), :]
        h0 = jax.nn.sigmoid(
            jnp.dot(e0, wt, preferred_element_type=jnp.float32)
            + jnp.dot(agg1, wb, preferred_element_type=jnp.float32) + ba)
        return jnp.tanh(
            jnp.dot(h0, wt, preferred_element_type=jnp.float32)
            + jnp.dot(aggf, wb, preferred_element_type=jnp.float32) + ba)

    side_u = e0u[...].astype(jnp.float32)
    item_graph = aggregate(side_u, e0i[...].astype(jnp.float32), e1i, e2i, r1i[...], r2i[...])
    user_graph = aggregate(item_graph, side_u, e1u, e2u, r1u[...], r2u[...])
    og[...] = jax.nn.sigmoid(jnp.sum(user_graph * item_graph, axis=1))

    ud = jax.nn.relu(jnp.dot(du[...], wdr[...],
                             preferred_element_type=jnp.float32) + bdr[...])
    idd = jax.nn.relu(jnp.dot(di[...], wdr[...],
                              preferred_element_type=jnp.float32) + bdr[...])
    nl = jax.nn.relu(
        jnp.dot(ud, w1[0:DIM, :], preferred_element_type=jnp.float32)
        + jnp.dot(idd, w1[DIM:2 * DIM, :], preferred_element_type=jnp.float32)
        + b1[...])
    nl = jax.nn.relu(jnp.dot(nl, w2[...],
                             preferred_element_type=jnp.float32) + b2[...])
    nl = jax.nn.relu(jnp.dot(nl, w3[...],
                             preferred_element_type=jnp.float32) + b3[...])
    lmul = ud * idd
    sd = (jnp.sum(lmul * wnm[:, 0:DIM], axis=1)
          + jnp.sum(nl * wnm[:, DIM:DIM + DIM // 2], axis=1) + bnm[0, 0])
    od[...] = jax.nn.sigmoid(sd)


def _tc_compute(bc, e0u, e1u, e2u, e0i, e1i, e2i, r1u, r2u, r1i, r2i,
                du, di, rel, wagg, bagg, wdr, bdr, w1, b1, w2, b2, w3, b3,
                wnm, bnm):
    f32 = jnp.float32
    nb = bc // BLK

    def blk(shape, imap):
        return pl.BlockSpec(shape, imap)

    row = lambda i: (i, 0)
    whole = lambda i: (0, 0)
    nmaj = lambda i: (0, i, 0)
    in_specs = [
        blk((BLK, DIM), row), blk((NNB, BLK, DIM), nmaj),
        blk((NNB * NNB, BLK, DIM), nmaj),
        blk((BLK, DIM), row), blk((NNB, BLK, DIM), nmaj),
        blk((NNB * NNB, BLK, DIM), nmaj),
        blk((BLK, NNB), row), blk((BLK, NNB * NNB), row),
        blk((BLK, NNB), row), blk((BLK, NNB * NNB), row),
        blk((BLK, DESC_DIM), row), blk((BLK, DESC_DIM), row),
        blk((32, DIM), whole), blk((2 * DIM, DIM), whole),
        blk((1, DIM), whole), blk((DESC_DIM, DIM), whole),
        blk((1, DIM), whole), blk((2 * DIM, 2 * DIM), whole),
        blk((1, 2 * DIM), whole), blk((2 * DIM, DIM), whole),
        blk((1, DIM), whole), blk((DIM, DIM // 2), whole),
        blk((1, DIM // 2), whole), blk((1, DIM + DIM // 2), whole),
        blk((1, 1), whole),
    ]
    out_specs = [pl.BlockSpec((BLK,), lambda i: (i,)),
                 pl.BlockSpec((BLK,), lambda i: (i,))]
    out_shape = [jax.ShapeDtypeStruct((bc,), f32),
                 jax.ShapeDtypeStruct((bc,), f32)]
    return pl.pallas_call(
        _tc_body, grid=(nb,), in_specs=in_specs, out_specs=out_specs,
        out_shape=out_shape,
    )(e0u, e1u, e2u, e0i, e1i, e2i, r1u, r2u, r1i, r2i, du, di,
      rel, wagg, bagg, wdr, bdr, w1, b1, w2, b2, w3, b3, wnm, bnm)


def kernel(user_index, item_index, adj_ent, adj_rel, ent, rel, desc_tab,
           W_agg, b_agg, W_dr, b_dr, W1, b1, W2, b2, W3, b3, W_nm, b_nm):
    num_ent = adj_ent.shape[0]
    adj_cat = jnp.concatenate(
        [adj_ent, adj_rel,
         jnp.zeros((num_ent, 128 - 2 * NNB), jnp.int32)], axis=1)
    bc = B // NCH
    sc_fn = _make_sc_gather(bc)
    ogs, ods = [], []
    for k in range(NCH):
        ui = lax.dynamic_slice_in_dim(user_index, k * bc, bc)
        ii = lax.dynamic_slice_in_dim(item_index, k * bc, bc)
        (rel1_u, rel2_u, rel1_i, rel2_i,
         e0u, e1u, e2u, e0i, e1i, e2i, du, di) = sc_fn(
            ui, ii, adj_cat, ent, desc_tab)
        og, od = _tc_compute(
            bc, e0u, e1u, e2u, e0i, e1i, e2i,
            rel1_u.reshape(bc, NNB), rel2_u.reshape(bc, NNB * NNB),
            rel1_i.reshape(bc, NNB), rel2_i.reshape(bc, NNB * NNB),
            du, di, rel, W_agg, b_agg.reshape(1, DIM), W_dr,
            b_dr.reshape(1, DIM), W1, b1.reshape(1, 2 * DIM), W2,
            b2.reshape(1, DIM), W3, b3.reshape(1, DIM // 2),
            W_nm.reshape(1, DIM + DIM // 2), b_nm.reshape(1, 1))
        ogs.append(og)
        ods.append(od)
    return jnp.concatenate(ogs), jnp.concatenate(ods)


# speedup vs baseline: 1.4675x; 1.1168x over previous
import functools

import jax
import jax.numpy as jnp
from jax import lax
from jax.experimental import pallas as pl
from jax.experimental.pallas import tpu as pltpu
from jax.experimental.pallas import tpu_sc as plsc

B = 4096
DIM = 128
DESC_DIM = 768
NNB = 8
NW = 32
BLK = 256
NCH = 1


def _make_sc_gather(bc):
    pb = bc // NW

    def sc_body(ui_hbm, ii_hbm, adj_cat, ent, desc_tab,
                rel1_u, rel2_u, rel1_i, rel2_i,
                e0_u, e1_u, e2_u, e0_i, e1_i, e2_i, d_u, d_i,
                idxall, relbuf, abuf, ebuf, ebuf2, dbuf, sem, wsem):
        nc = 2
        wid = lax.axis_index("s") * nc + lax.axis_index("c")
        base = pl.multiple_of(wid * pb, pb)

        def extract(dst_ref, dst0, col):
            def ex(j, _):
                v = abuf[j, col:col + 16]
                dst_ref[pl.ds(pl.multiple_of(dst0 + 8 * j, 8), 16)] = v
                return 0
            lax.fori_loop(0, pb, ex, 0)

        def do_side(base_hbm, rel1_o, rel2_o, e0_o, e1_o, e2_o, d_o):
            pltpu.sync_copy(base_hbm.at[pl.ds(base, pb)],
                            idxall.at[pl.ds(0, pb)])
            pltpu.async_copy(adj_cat.at[idxall.at[pl.ds(0, pb)]], abuf,
                             sem).wait()
            extract(idxall, pb, 0)
            extract(relbuf, 0, 8)

            def hop2(c, _):
                off = pl.multiple_of(pb + pb * c, pb)
                pltpu.async_copy(adj_cat.at[idxall.at[pl.ds(off, pb)]],
                                 abuf, sem).wait()
                extract(idxall, pb * (1 + NNB) + pb * NNB * c, 0)
                extract(relbuf, pb * NNB + pb * NNB * c, 8)
                return 0
            lax.fori_loop(0, NNB, hop2, 0)
            pltpu.sync_copy(
                relbuf.at[pl.ds(0, pb * NNB)],
                rel1_o.at[pl.ds(pl.multiple_of(wid * pb * NNB, pb),
                                pb * NNB)])
            pltpu.sync_copy(
                relbuf.at[pl.ds(pb * NNB, pb * NNB * NNB)],
                rel2_o.at[pl.ds(pl.multiple_of(wid * pb * NNB * NNB, pb),
                                pb * NNB * NNB)])

            pltpu.async_copy(ent.at[idxall.at[pl.ds(0, pb)]],
                             ebuf.at[pl.ds(0, pb), :], sem).wait()
            pltpu.sync_copy(ebuf.at[pl.ds(0, pb), :],
                            e0_o.at[pl.ds(base, pb)])

            def drain2():
                pltpu.make_async_copy(ent.at[pl.ds(0, 128)], ebuf,
                                      wsem).wait()
                pltpu.make_async_copy(ent.at[pl.ds(0, 128)], ebuf2,
                                      wsem).wait()

            def g1(c2, _):
                @pl.when(c2 > 0)
                def _():
                    drain2()
                for k, buf in ((0, ebuf), (1, ebuf2)):
                    c = 2 * c2 + k
                    off = pl.multiple_of(pb + 128 * c, 8)
                    pltpu.async_copy(ent.at[idxall.at[pl.ds(off, 128)]],
                                     buf, sem).wait()
                    for t in range(16):
                        pltpu.async_copy(buf.at[pl.ds(NNB * t, NNB), :],
                                         e1_o.at[:, base + 16 * c + t, :],
                                         wsem)
                return 0
            lax.fori_loop(0, pb // 32, g1, 0)
            drain2()

            def g2(c2, _):
                @pl.when(c2 > 0)
                def _():
                    drain2()
                for k, buf in ((0, ebuf), (1, ebuf2)):
                    c = 2 * c2 + k
                    off = pl.multiple_of(pb * (1 + NNB) + 128 * c, 8)
                    pltpu.async_copy(ent.at[idxall.at[pl.ds(off, 128)]],
                                     buf, sem).wait()
                    pltpu.async_copy(buf.at[pl.ds(0, 64), :],
                                     e2_o.at[:, base + 2 * c, :], wsem)
                    pltpu.async_copy(buf.at[pl.ds(64, 64), :],
                                     e2_o.at[:, base + 2 * c + 1, :], wsem)
                return 0
            lax.fori_loop(0, pb // 4, g2, 0)
            drain2()

            def gd(c, _):
                off = pl.multiple_of(16 * c, 16)
                pltpu.async_copy(desc_tab.at[idxall.at[pl.ds(off, 16)]],
                                 dbuf, sem).wait()
                dst = pl.multiple_of(wid * pb + 16 * c, 16)
                pltpu.sync_copy(dbuf, d_o.at[pl.ds(dst, 16)])
                return 0
            lax.fori_loop(0, pb // 16, gd, 0)

        do_side(ui_hbm, rel1_u, rel2_u, e0_u, e1_u, e2_u, d_u)
        do_side(ii_hbm, rel1_i, rel2_i, e0_i, e1_i, e2_i, d_i)

    i32, f32, bf16 = jnp.int32, jnp.float32, jnp.bfloat16
    out_type = [
        jax.ShapeDtypeStruct((bc * NNB,), i32),
        jax.ShapeDtypeStruct((bc * NNB * NNB,), i32),
        jax.ShapeDtypeStruct((bc * NNB,), i32),
        jax.ShapeDtypeStruct((bc * NNB * NNB,), i32),
        jax.ShapeDtypeStruct((bc, DIM), f32),
        jax.ShapeDtypeStruct((NNB, bc, DIM), f32),
        jax.ShapeDtypeStruct((NNB * NNB, bc, DIM), f32),
        jax.ShapeDtypeStruct((bc, DIM), f32),
        jax.ShapeDtypeStruct((NNB, bc, DIM), f32),
        jax.ShapeDtypeStruct((NNB * NNB, bc, DIM), f32),
        jax.ShapeDtypeStruct((bc, DESC_DIM), f32),
        jax.ShapeDtypeStruct((bc, DESC_DIM), f32),
    ]
    scratch = [
        pltpu.VMEM((pb * (1 + NNB + NNB * NNB) + 16,), i32),
        pltpu.VMEM((pb * (NNB + NNB * NNB) + 16,), i32),
        pltpu.VMEM((pb, 128), i32),
        pltpu.VMEM((128, DIM), f32),
        pltpu.VMEM((128, DIM), f32),
        pltpu.VMEM((16, DESC_DIM), f32),
        pltpu.SemaphoreType.DMA,
        pltpu.SemaphoreType.DMA,
    ]
    mesh = plsc.VectorSubcoreMesh(core_axis_name="c", subcore_axis_name="s")
    return pl.kernel(sc_body, out_type=out_type, mesh=mesh,
                     scratch_types=scratch)


def _tc_body(e0u, e1u, e2u, e0i, e1i, e2i, r1u, r2u, r1i, r2i, du, di,
             rel, wagg, bagg, wdr, bdr, w1, b1, w2, b2, w3, b3, wnm, bnm,
             og, od):
    wt = wagg[0:DIM, :]
    wb = wagg[DIM:2 * DIM, :]
    ba = bagg[...]
    rel_m = rel[...]

    def gather_scores(s_rel, idx):
        out = jnp.zeros(idx.shape, jnp.float32)
        for k in range(32):
            out = out + jnp.where(idx == k, s_rel[:, k:k + 1], 0.0)
        return out

    def softmax_rows(s):
        m = jnp.max(s, axis=-1, keepdims=True)
        e = jnp.exp(s - m)
        return e / jnp.sum(e, axis=-1, keepdims=True)

    def aggregate(side, e0, e1_ref, e2_ref, r1, r2):
        s_rel = lax.dot_general(side, rel_m, (((1,), (1,)), ((), ())),
                                preferred_element_type=jnp.float32)
        s2 = gather_scores(s_rel, r2)
        e1s = [e1_ref[i, :, :].astype(jnp.float32) for i in range(NNB)]
        aggs = []
        for i in range(NNB):
            w2g = softmax_rows(s2[:, NNB * i:NNB * (i + 1)])
            a = w2g[:, 0:1] * e2_ref[NNB * i, :, :].astype(jnp.float32)
            for n in range(1, NNB):
                a = a + w2g[:, n:n + 1] * e2_ref[NNB * i + n, :, :].astype(jnp.float32)
            aggs.append(a)
        x1 = jnp.concatenate(e1s, axis=0)
        a1 = jnp.concatenate(aggs, axis=0)
        h1 = jax.nn.sigmoid(
            jnp.dot(x1, wt, preferred_element_type=jnp.float32)
            + jnp.dot(a1, wb, preferred_element_type=jnp.float32) + ba)
        w1a = softmax_rows(gather_scores(s_rel, r1))
        agg1 = w1a[:, 0:1] * e1s[0]
        aggf = w1a[:, 0:1] * h1[0:BLK, :]
        for i in range(1, NNB):
            agg1 = agg1 + w1a[:, i:i + 1] * e1s[i]
            aggf = aggf + w1a[:, i:i + 1] * h1[BLK * i:BLK * (i + 1), :]
        h0 = jax.nn.sigmoid(
            jnp.dot(e0, wt, preferred_element_type=jnp.float32)
            + jnp.dot(agg1, wb, preferred_element_type=jnp.float32) + ba)
        return jnp.tanh(
            jnp.dot(h0, wt, preferred_element_type=jnp.float32)
            + jnp.dot(aggf, wb, preferred_element_type=jnp.float32) + ba)

    side_u = e0u[...].astype(jnp.float32)
    item_graph = aggregate(side_u, e0i[...].astype(jnp.float32), e1i, e2i, r1i[...], r2i[...])
    user_graph = aggregate(item_graph, side_u, e1u, e2u, r1u[...], r2u[...])
    og[...] = jax.nn.sigmoid(jnp.sum(user_graph * item_graph, axis=1))

    ud = jax.nn.relu(jnp.dot(du[...], wdr[...],
                             preferred_element_type=jnp.float32) + bdr[...])
    idd = jax.nn.relu(jnp.dot(di[...], wdr[...],
                              preferred_element_type=jnp.float32) + bdr[...])
    nl = jax.nn.relu(
        jnp.dot(ud, w1[0:DIM, :], preferred_element_type=jnp.float32)
        + jnp.dot(idd, w1[DIM:2 * DIM, :], preferred_element_type=jnp.float32)
        + b1[...])
    nl = jax.nn.relu(jnp.dot(nl, w2[...],
                             preferred_element_type=jnp.float32) + b2[...])
    nl = jax.nn.relu(jnp.dot(nl, w3[...],
                             preferred_element_type=jnp.float32) + b3[...])
    lmul = ud * idd
    sd = (jnp.sum(lmul * wnm[:, 0:DIM], axis=1)
          + jnp.sum(nl * wnm[:, DIM:DIM + DIM // 2], axis=1) + bnm[0, 0])
    od[...] = jax.nn.sigmoid(sd)


def _tc_compute(bc, e0u, e1u, e2u, e0i, e1i, e2i, r1u, r2u, r1i, r2i,
                du, di, rel, wagg, bagg, wdr, bdr, w1, b1, w2, b2, w3, b3,
                wnm, bnm):
    f32 = jnp.float32
    nb = bc // BLK

    def blk(shape, imap):
        return pl.BlockSpec(shape, imap)

    row = lambda i: (i, 0)
    whole = lambda i: (0, 0)
    nmaj = lambda i: (0, i, 0)
    in_specs = [
        blk((BLK, DIM), row), blk((NNB, BLK, DIM), nmaj),
        blk((NNB * NNB, BLK, DIM), nmaj),
        blk((BLK, DIM), row), blk((NNB, BLK, DIM), nmaj),
        blk((NNB * NNB, BLK, DIM), nmaj),
        blk((BLK, NNB), row), blk((BLK, NNB * NNB), row),
        blk((BLK, NNB), row), blk((BLK, NNB * NNB), row),
        blk((BLK, DESC_DIM), row), blk((BLK, DESC_DIM), row),
        blk((32, DIM), whole), blk((2 * DIM, DIM), whole),
        blk((1, DIM), whole), blk((DESC_DIM, DIM), whole),
        blk((1, DIM), whole), blk((2 * DIM, 2 * DIM), whole),
        blk((1, 2 * DIM), whole), blk((2 * DIM, DIM), whole),
        blk((1, DIM), whole), blk((DIM, DIM // 2), whole),
        blk((1, DIM // 2), whole), blk((1, DIM + DIM // 2), whole),
        blk((1, 1), whole),
    ]
    out_specs = [pl.BlockSpec((BLK,), lambda i: (i,)),
                 pl.BlockSpec((BLK,), lambda i: (i,))]
    out_shape = [jax.ShapeDtypeStruct((bc,), f32),
                 jax.ShapeDtypeStruct((bc,), f32)]
    return pl.pallas_call(
        _tc_body, grid=(nb,), in_specs=in_specs, out_specs=out_specs,
        out_shape=out_shape,
    )(e0u, e1u, e2u, e0i, e1i, e2i, r1u, r2u, r1i, r2i, du, di,
      rel, wagg, bagg, wdr, bdr, w1, b1, w2, b2, w3, b3, wnm, bnm)


def kernel(user_index, item_index, adj_ent, adj_rel, ent, rel, desc_tab,
           W_agg, b_agg, W_dr, b_dr, W1, b1, W2, b2, W3, b3, W_nm, b_nm):
    num_ent = adj_ent.shape[0]
    adj_cat = jnp.concatenate(
        [adj_ent, adj_rel,
         jnp.zeros((num_ent, 128 - 2 * NNB), jnp.int32)], axis=1)
    bc = B // NCH
    sc_fn = _make_sc_gather(bc)
    ogs, ods = [], []
    for k in range(NCH):
        ui = lax.dynamic_slice_in_dim(user_index, k * bc, bc)
        ii = lax.dynamic_slice_in_dim(item_index, k * bc, bc)
        (rel1_u, rel2_u, rel1_i, rel2_i,
         e0u, e1u, e2u, e0i, e1i, e2i, du, di) = sc_fn(
            ui, ii, adj_cat, ent, desc_tab)
        og, od = _tc_compute(
            bc, e0u, e1u, e2u, e0i, e1i, e2i,
            rel1_u.reshape(bc, NNB), rel2_u.reshape(bc, NNB * NNB),
            rel1_i.reshape(bc, NNB), rel2_i.reshape(bc, NNB * NNB),
            du, di, rel, W_agg, b_agg.reshape(1, DIM), W_dr,
            b_dr.reshape(1, DIM), W1, b1.reshape(1, 2 * DIM), W2,
            b2.reshape(1, DIM), W3, b3.reshape(1, DIM // 2),
            W_nm.reshape(1, DIM + DIM // 2), b_nm.reshape(1, 1))
        ogs.append(og)
        ods.append(od)
    return jnp.concatenate(ogs), jnp.concatenate(ods)
